# baseline (speedup 1.0000x reference)
import functools

import jax
import jax.numpy as jnp
from jax import lax
from jax.experimental import pallas as pl
from jax.experimental.pallas import tpu as pltpu
from jax.experimental.pallas import tpu_sc as plsc

NC, NS = 2, 16
NW = NC * NS
D = 64
CH = 512
SB = 12800
NSLOT = 3


@functools.partial(jax.jit, static_argnums=(2,))
def _lookup(pos_flat, table, B):
    per_w = B // NW
    n_sb = per_w // SB
    n_ch = SB // CH

    mesh = plsc.VectorSubcoreMesh(
        core_axis_name="c", subcore_axis_name="s",
        num_cores=NC, num_subcores=NS)

    @functools.partial(
        pl.kernel,
        out_type=jax.ShapeDtypeStruct((B, D), jnp.float32),
        mesh=mesh,
        scratch_types=[
            pltpu.VMEM((SB,), jnp.int32),
            pltpu.VMEM((NSLOT, CH, D), jnp.float32),
            pltpu.SemaphoreType.DMA((NSLOT,)),
            pltpu.SemaphoreType.DMA((NSLOT,)),
        ],
        compiler_params=pltpu.CompilerParams(use_tc_tiling_on_sc=False),
    )
    def k(pos_hbm, tab_hbm, out_hbm, idx_v, rows_v, gsem, osem):
        wid = lax.axis_index("s") * NC + lax.axis_index("c")
        base = wid * per_w

        def fire_gather(g, s):
            pltpu.async_copy(
                tab_hbm.at[idx_v.at[pl.ds(g * CH, CH)]], rows_v.at[s],
                gsem.at[s])

        def wait_gather(g, s):
            pltpu.make_async_copy(
                tab_hbm.at[idx_v.at[pl.ds(g * CH, CH)]], rows_v.at[s],
                gsem.at[s]).wait()

        def fire_write(sb_base, g, s):
            pltpu.async_copy(
                rows_v.at[s], out_hbm.at[pl.ds(sb_base + g * CH, CH)],
                osem.at[s])

        def wait_write(sb_base, g, s):
            pltpu.make_async_copy(
                rows_v.at[s], out_hbm.at[pl.ds(sb_base + g * CH, CH)],
                osem.at[s]).wait()

        def sb_body(sbi, carry):
            sb_base = base + sbi * SB
            pltpu.sync_copy(pos_hbm.at[pl.ds(sb_base, SB)], idx_v)

            fire_gather(0, 0)
            fire_gather(1, 1)
            wait_gather(0, 0)
            fire_write(sb_base, 0, 0)
            fire_gather(2, 2)
            wait_gather(1, 1)
            fire_write(sb_base, 1, 1)

            def step(g, carry):
                sn = lax.rem(g + 1, NSLOT)
                sc = lax.rem(g, NSLOT)
                wait_write(sb_base, g - 2, sn)
                fire_gather(g + 1, sn)
                wait_gather(g, sc)
                fire_write(sb_base, g, sc)
                return carry

            lax.fori_loop(2, n_ch - 1, step, carry)

            g_last = n_ch - 1
            s_last = lax.rem(g_last, NSLOT)
            wait_gather(g_last, s_last)
            fire_write(sb_base, g_last, s_last)
            wait_write(sb_base, g_last - 2, lax.rem(g_last - 2, NSLOT))
            wait_write(sb_base, g_last - 1, lax.rem(g_last - 1, NSLOT))
            wait_write(sb_base, g_last, s_last)
            return carry

        lax.fori_loop(0, n_sb, sb_body, 0)

    return k(pos_flat, table)


def kernel(positions, table):
    batch, seq = positions.shape
    b = batch * seq
    pos_flat = positions.reshape(b).astype(jnp.int32)
    out = _lookup(pos_flat, table, b)
    return out.reshape(batch, seq, D)


# speedup vs baseline: 1.2630x; 1.2630x over previous
import functools

import jax
import jax.numpy as jnp
from jax import lax
from jax.experimental import pallas as pl
from jax.experimental.pallas import tpu as pltpu
from jax.experimental.pallas import tpu_sc as plsc

NC, NS = 2, 16
NW = NC * NS
D = 64
V = 200
CH = 512
SB = 12800
NSLOT = 3
U = 16


@functools.partial(jax.jit, static_argnums=(2,))
def _lookup(pos_flat, tab_flat, B):
    per_w = B // NW
    n_sb = per_w // SB
    n_ch = SB // CH

    mesh = plsc.VectorSubcoreMesh(
        core_axis_name="c", subcore_axis_name="s",
        num_cores=NC, num_subcores=NS)

    @functools.partial(
        pl.kernel,
        out_type=jax.ShapeDtypeStruct((B * D,), jnp.float32),
        mesh=mesh,
        scratch_types=[
            pltpu.VMEM((V * D,), jnp.float32),
            pltpu.VMEM((SB,), jnp.int32),
            pltpu.VMEM((NSLOT, CH * D), jnp.float32),
            pltpu.SemaphoreType.DMA((NSLOT,)),
        ],
        compiler_params=pltpu.CompilerParams(use_tc_tiling_on_sc=False),
    )
    def k(pos_hbm, tab_hbm, out_hbm, tab_v, idx_v, rows_v, osem):
        wid = lax.axis_index("s") * NC + lax.axis_index("c")
        base = wid * per_w

        pltpu.sync_copy(tab_hbm, tab_v)

        def wait_write(sb_base, g, s):
            pltpu.make_async_copy(
                rows_v.at[s],
                out_hbm.at[pl.ds((sb_base + g * CH) * D, CH * D)],
                osem.at[s]).wait()

        def sb_body(sbi, carry):
            sb_base = base + sbi * SB
            pltpu.sync_copy(pos_hbm.at[pl.ds(sb_base, SB)], idx_v)

            def g_body(g, carry):
                s = lax.rem(g, NSLOT)

                @pl.when(g >= NSLOT)
                def _():
                    wait_write(sb_base, g - NSLOT, s)

                slot = rows_v.at[s]
                goff = g * CH

                def row_body(r, carry):
                    rbase = r * U
                    iv = idx_v[pl.ds(goff + rbase, U)] * D
                    for u in range(U):
                        tb = iv[u]
                        ob = (rbase + u) * D
                        for kk in range(D // 16):
                            slot[pl.ds(ob + kk * 16, 16)] = (
                                tab_v[pl.ds(tb + kk * 16, 16)])
                    return carry

                lax.fori_loop(0, CH // U, row_body, carry)

                pltpu.async_copy(
                    slot,
                    out_hbm.at[pl.ds((sb_base + goff) * D, CH * D)],
                    osem.at[s])
                return carry

            lax.fori_loop(0, n_ch, g_body, carry)

            for g in (n_ch - 3, n_ch - 2, n_ch - 1):
                wait_write(sb_base, g, lax.rem(g, NSLOT))
            return carry

        lax.fori_loop(0, n_sb, sb_body, 0)

    return k(pos_flat, tab_flat)


def kernel(positions, table):
    batch, seq = positions.shape
    b = batch * seq
    pos_flat = positions.reshape(b).astype(jnp.int32)
    out = _lookup(pos_flat, table.reshape(V * D), b)
    return out.reshape(batch, seq, D)
